# baseline (speedup 1.0000x reference)
import functools

import jax
import jax.numpy as jnp
from jax import lax
from jax.experimental import pallas as pl
from jax.experimental.pallas import tpu as pltpu
from jax.experimental.pallas import tpu_sc as plsc

D_MODEL = 768
N_EXPERTS = 16
T = 2048
NC = 2
NS = 16
BLK = 256
NBLK = 32
S = NBLK * BLK
TPW = T // NS
SPW = S // (NC * NS)
ROWS_CHUNK = 64
CPW = T // (NC * NS)
CCH = 32

_mesh = plsc.VectorSubcoreMesh(
    core_axis_name="c", subcore_axis_name="s", num_cores=NC, num_subcores=NS
)


def _gate_kernel(x_ref, wg_ref, bg_ref, out_ref):
    out_ref[...] = (
        jnp.dot(x_ref[...], wg_ref[...], preferred_element_type=jnp.float32)
        + bg_ref[...]
    ).T


def _lane_iota():
    return lax.iota(jnp.int32, 16)


_TAKE_DNUMS = lax.GatherDimensionNumbers(
    offset_dims=(), collapsed_slice_dims=(0,), start_index_map=(0,)
)


def _vreg_take(vec, lidx):
    return lax.gather(
        vec,
        lidx[:, None],
        _TAKE_DNUMS,
        slice_sizes=(1,),
        mode=lax.GatherScatterMode.PROMISE_IN_BOUNDS,
    )


def _b2i(mask):
    return jnp.where(mask, jnp.ones((16,), jnp.int32), jnp.zeros((16,), jnp.int32))


def _splat(vec, l):
    return _vreg_take(vec, jnp.full((16,), l, jnp.int32))


def _prefix_sum(vec):
    lanes = _lane_iota()
    cs = vec
    for k in (1, 2, 4, 8):
        sh = _vreg_take(cs, jnp.clip(lanes - k, 0, 15))
        cs = cs + jnp.where(lanes >= k, sh, 0)
    return cs


def _route_gather_body(
    logits_hbm, x_hbm,
    blk_e_hbm, p1_hbm, p2_hbm, w1_hbm, w2_hbm, gx_hbm, cntx_hbm,
    lg_v, e1_v, e2_v, w1_v, w2_v, p1_v, p2_v, cnt_v, allcnt_v,
    blkexp_v, idx2_v, rows_v, sem,
):
    c = lax.axis_index("c")
    s = lax.axis_index("s")
    tbase = s * TPW
    lanes = _lane_iota()

    for e in range(N_EXPERTS):
        pltpu.sync_copy(
            logits_hbm.at[pl.ds(e * T + tbase, TPW)], lg_v.at[pl.ds(e * TPW, TPW)]
        )

    def tile_route(tile, cnt):
        m1 = jnp.full((16,), -1e30, jnp.float32)
        m2 = jnp.full((16,), -1e30, jnp.float32)
        e1 = jnp.zeros((16,), jnp.int32)
        e2 = jnp.zeros((16,), jnp.int32)
        for e in range(N_EXPERTS):
            col = jnp.full((16,), e, jnp.int32)
            v = lg_v[pl.ds(e * TPW + tile * 16, 16)]
            gt1 = v > m1
            gt2 = v > m2
            m2n = jnp.where(gt1, m1, jnp.where(gt2, v, m2))
            e2n = jnp.where(gt1, e1, jnp.where(gt2, col, e2))
            m1 = jnp.where(gt1, v, m1)
            e1 = jnp.where(gt1, col, e1)
            m2, e2 = m2n, e2n
        w1t = 1.0 / (1.0 + jnp.exp(m2 - m1))
        off = pl.ds(tile * 16, 16)
        e1_v[off] = e1
        e2_v[off] = e2
        w1_v[off] = w1t
        w2_v[off] = 1.0 - w1t
        for l in range(16):
            cnt = cnt + _b2i(lanes == _splat(e1, l))
            cnt = cnt + _b2i(lanes == _splat(e2, l))
        return cnt

    cnt = lax.fori_loop(0, TPW // 16, tile_route, jnp.zeros((16,), jnp.int32))

    cnt_v[...] = cnt
    pltpu.sync_copy(cnt_v, cntx_hbm.at[s])
    plsc.subcore_barrier()
    pltpu.sync_copy(cntx_hbm, allcnt_v)
    tot = jnp.zeros((16,), jnp.int32)
    base = jnp.zeros((16,), jnp.int32)
    for w in range(NS):
        row = allcnt_v[w]
        before = _b2i(jnp.full((16,), w, jnp.int32) < s)
        base = base + row * before
        tot = tot + row
    nblk = (tot + (BLK - 1)) >> 8
    blkcum_excl = _prefix_sum(nblk) - nblk
    goff = blkcum_excl * BLK + base

    @pl.when(s == 0)
    def _blockmap():
        for half in range(NBLK // 16):
            gv = lanes + half * 16
            acc = jnp.zeros((16,), jnp.int32)
            for e in range(N_EXPERTS):
                acc = acc + _b2i(gv >= _splat(blkcum_excl, e))
            blkexp_v[pl.ds(half * 16, 16)] = jnp.clip(acc - 1, 0, N_EXPERTS - 1)
        pltpu.sync_copy(blkexp_v, blk_e_hbm)

    def tile_pos(tile, loc):
        off = pl.ds(tile * 16, 16)
        for ev_ref, pv_ref in ((e1_v, p1_v), (e2_v, p2_v)):
            ev = ev_ref[off]
            base = _vreg_take(loc, ev)
            rank = jnp.zeros((16,), jnp.int32)
            hist = jnp.zeros((16,), jnp.int32)
            for l in range(16):
                sp = _splat(ev, l)
                rank = rank + _b2i((ev == sp) & (lanes > l))
                hist = hist + _b2i(lanes == sp)
            pv_ref[off] = base + rank
            loc = loc + hist
        return loc

    lax.fori_loop(0, TPW // 16, tile_pos, goff)

    pltpu.sync_copy(p1_v, p1_hbm.at[pl.ds(tbase, TPW)])
    pltpu.sync_copy(p2_v, p2_hbm.at[pl.ds(tbase, TPW)])
    pltpu.sync_copy(w1_v, w1_hbm.at[pl.ds(tbase, TPW)])
    pltpu.sync_copy(w2_v, w2_hbm.at[pl.ds(tbase, TPW)])

    half = c * (TPW // 2)
    pltpu.sync_copy(x_hbm.at[pl.ds(tbase + half, TPW // 2)], rows_v)
    for i in range(TPW // 2 // 16):
        off = pl.ds(i * 16, 16)
        idx2_v[0, off] = p1_v[pl.ds(half + i * 16, 16)]
        idx2_v[1, off] = p2_v[pl.ds(half + i * 16, 16)]
    pltpu.async_copy(rows_v, gx_hbm.at[idx2_v.at[0]], sem).wait()
    pltpu.async_copy(rows_v, gx_hbm.at[idx2_v.at[1]], sem).wait()


_route_gather = functools.partial(
    pl.kernel,
    _route_gather_body,
    mesh=_mesh,
    out_type=(
        jax.ShapeDtypeStruct((NBLK,), jnp.int32),
        jax.ShapeDtypeStruct((T,), jnp.int32),
        jax.ShapeDtypeStruct((T,), jnp.int32),
        jax.ShapeDtypeStruct((T,), jnp.float32),
        jax.ShapeDtypeStruct((T,), jnp.float32),
        jax.ShapeDtypeStruct((S, D_MODEL // 2), jnp.int32),
        jax.ShapeDtypeStruct((NS, 16), jnp.int32),
    ),
    scratch_types=[
        pltpu.VMEM((TPW * N_EXPERTS,), jnp.float32),
        pltpu.VMEM((TPW,), jnp.int32),
        pltpu.VMEM((TPW,), jnp.int32),
        pltpu.VMEM((TPW,), jnp.float32),
        pltpu.VMEM((TPW,), jnp.float32),
        pltpu.VMEM((TPW,), jnp.int32),
        pltpu.VMEM((TPW,), jnp.int32),
        pltpu.VMEM((16,), jnp.int32),
        pltpu.VMEM((NS, 16), jnp.int32),
        pltpu.VMEM((NBLK,), jnp.int32),
        pltpu.VMEM((2, TPW // 2), jnp.int32),
        pltpu.VMEM((TPW // 2, D_MODEL // 2), jnp.int32),
        pltpu.SemaphoreType.DMA,
    ],
)()


def _expert_mm_kernel(be_ref, gx_ref, we_ref, beb_ref, y_ref):
    y_ref[...] = (
        jnp.dot(gx_ref[...], we_ref[0], preferred_element_type=jnp.float32)
        + beb_ref[0]
    )


def _combine_body(
    y_hbm, p1_hbm, p2_hbm, w1_hbm, w2_hbm, out_hbm,
    p1_v, p2_v, w1_v, w2_v, a_v, b_v, o_v, sem,
):
    wid = lax.axis_index("c") * NS + lax.axis_index("s")
    tb = wid * CPW
    pltpu.sync_copy(w1_hbm.at[pl.ds(tb, CPW)], w1_v)
    pltpu.sync_copy(w2_hbm.at[pl.ds(tb, CPW)], w2_v)
    for ch in range(CPW // CCH):
        pltpu.sync_copy(p1_hbm.at[pl.ds(tb + ch * CCH, CCH)], p1_v.at[ch])
        pltpu.sync_copy(p2_hbm.at[pl.ds(tb + ch * CCH, CCH)], p2_v.at[ch])
        for i in range(CCH // 16):
            off = pl.ds(i * 16, 16)
            p1_v[ch, off] = jnp.clip(p1_v[ch, off], 0, S - 1)
            p2_v[ch, off] = jnp.clip(p2_v[ch, off], 0, S - 1)

    def chunk_body(ch, _):
        pltpu.async_copy(y_hbm.at[p1_v.at[ch]], a_v, sem).wait()
        pltpu.async_copy(y_hbm.at[p2_v.at[ch]], b_v, sem).wait()

        def grp_body(g, _):
            w16_1 = w1_v[pl.ds(ch * CCH + g * 16, 16)]
            w16_2 = w2_v[pl.ds(ch * CCH + g * 16, 16)]
            for l in range(16):
                i = g * 16 + l
                lidx = jnp.full((16,), l, jnp.int32)
                w1b = _vreg_take(w16_1, lidx)
                w2b = _vreg_take(w16_2, lidx)
                for v in range(D_MODEL // 16):
                    off = pl.ds(v * 16, 16)
                    o_v[i, off] = w1b * a_v[i, off] + w2b * b_v[i, off]
            return 0

        lax.fori_loop(0, CCH // 16, grp_body, 0)
        pltpu.sync_copy(o_v, out_hbm.at[pl.ds(tb + ch * CCH, CCH)])
        return 0

    lax.fori_loop(0, CPW // CCH, chunk_body, 0)


_combine = functools.partial(
    pl.kernel,
    _combine_body,
    mesh=_mesh,
    out_type=jax.ShapeDtypeStruct((T, D_MODEL), jnp.float32),
    scratch_types=[
        pltpu.VMEM((CPW // CCH, CCH), jnp.int32),
        pltpu.VMEM((CPW // CCH, CCH), jnp.int32),
        pltpu.VMEM((CPW,), jnp.float32),
        pltpu.VMEM((CPW,), jnp.float32),
        pltpu.VMEM((CCH, D_MODEL), jnp.float32),
        pltpu.VMEM((CCH, D_MODEL), jnp.float32),
        pltpu.VMEM((CCH, D_MODEL), jnp.float32),
        pltpu.SemaphoreType.DMA,
    ],
)()


def kernel(inputs, Wg, bg, We, be):
    x2 = inputs.reshape(T, D_MODEL)
    bg2 = bg.reshape(1, N_EXPERTS)
    be3 = be.reshape(N_EXPERTS, 1, D_MODEL)

    logits = pl.pallas_call(
        _gate_kernel,
        out_shape=jax.ShapeDtypeStruct((N_EXPERTS, T), jnp.float32),
    )(x2, Wg, bg2)

    x_bf_i32 = jax.lax.bitcast_convert_type(
        x2.astype(jnp.bfloat16).reshape(T, D_MODEL // 2, 2), jnp.int32
    )
    blk_e, p1, p2, w1, w2, gx_i32, _cx = _route_gather(logits.reshape(-1), x_bf_i32)
    gx = jax.lax.bitcast_convert_type(gx_i32, jnp.bfloat16).reshape(S, D_MODEL)

    grid_spec = pltpu.PrefetchScalarGridSpec(
        num_scalar_prefetch=1,
        grid=(NBLK,),
        in_specs=[
            pl.BlockSpec((BLK, D_MODEL), lambda g, be_r: (g, 0)),
            pl.BlockSpec((1, D_MODEL, D_MODEL), lambda g, be_r: (be_r[g], 0, 0)),
            pl.BlockSpec((1, 1, D_MODEL), lambda g, be_r: (be_r[g], 0, 0)),
        ],
        out_specs=pl.BlockSpec((BLK, D_MODEL), lambda g, be_r: (g, 0)),
    )
    y = pl.pallas_call(
        _expert_mm_kernel,
        grid_spec=grid_spec,
        out_shape=jax.ShapeDtypeStruct((S, D_MODEL), jnp.float32),
    )(blk_e, gx, We.astype(jnp.bfloat16), be3)

    out = _combine(y, p1, p2, w1, w2)
    return out.reshape(inputs.shape)


# speedup vs baseline: 2.4016x; 2.4016x over previous
import functools

import jax
import jax.numpy as jnp
from jax import lax
from jax.experimental import pallas as pl
from jax.experimental.pallas import tpu as pltpu
from jax.experimental.pallas import tpu_sc as plsc

D_MODEL = 768
N_EXPERTS = 16
T = 2048
NC = 2
NS = 16
BLK = 256
NBLK = 32
S = NBLK * BLK
TPW = T // NS
SPW = S // (NC * NS)
ROWS_CHUNK = 64
CPW = T // (NC * NS)
CCH = 32

_mesh = plsc.VectorSubcoreMesh(
    core_axis_name="c", subcore_axis_name="s", num_cores=NC, num_subcores=NS
)


def _gate_kernel(x_ref, wg_ref, bg_ref, out_ref):
    out_ref[...] = (
        jnp.dot(x_ref[...], wg_ref[...], preferred_element_type=jnp.float32)
        + bg_ref[...]
    ).T


def _lane_iota():
    return lax.iota(jnp.int32, 16)


_TAKE_DNUMS = lax.GatherDimensionNumbers(
    offset_dims=(), collapsed_slice_dims=(0,), start_index_map=(0,)
)


def _vreg_take(vec, lidx):
    return lax.gather(
        vec,
        lidx[:, None],
        _TAKE_DNUMS,
        slice_sizes=(1,),
        mode=lax.GatherScatterMode.PROMISE_IN_BOUNDS,
    )


def _b2i(mask):
    return jnp.where(mask, jnp.ones((16,), jnp.int32), jnp.zeros((16,), jnp.int32))


def _splat(vec, l):
    return _vreg_take(vec, jnp.full((16,), l, jnp.int32))


def _prefix_sum(vec):
    lanes = _lane_iota()
    cs = vec
    for k in (1, 2, 4, 8):
        sh = _vreg_take(cs, jnp.clip(lanes - k, 0, 15))
        cs = cs + jnp.where(lanes >= k, sh, 0)
    return cs


def _route_gather_body(
    logits_hbm, x_hbm,
    blk_e_hbm, p1_hbm, p2_hbm, w1_hbm, w2_hbm, gx_hbm, cntx_hbm,
    lg_v, e1_v, e2_v, w1_v, w2_v, p1_v, p2_v, cnt_v, allcnt_v,
    blkexp_v, idx2_v, rows_v, sem, sem2,
):
    c = lax.axis_index("c")
    s = lax.axis_index("s")
    tbase = s * TPW
    lanes = _lane_iota()

    cps = [
        pltpu.async_copy(
            logits_hbm.at[pl.ds(e * T + tbase, TPW)], lg_v.at[pl.ds(e * TPW, TPW)], sem
        )
        for e in range(N_EXPERTS)
    ]
    for cp in cps:
        cp.wait()

    def tile_route(tile, cnt):
        m1 = jnp.full((16,), -1e30, jnp.float32)
        m2 = jnp.full((16,), -1e30, jnp.float32)
        e1 = jnp.zeros((16,), jnp.int32)
        e2 = jnp.zeros((16,), jnp.int32)
        for e in range(N_EXPERTS):
            col = jnp.full((16,), e, jnp.int32)
            v = lg_v[pl.ds(e * TPW + tile * 16, 16)]
            gt1 = v > m1
            gt2 = v > m2
            m2n = jnp.where(gt1, m1, jnp.where(gt2, v, m2))
            e2n = jnp.where(gt1, e1, jnp.where(gt2, col, e2))
            m1 = jnp.where(gt1, v, m1)
            e1 = jnp.where(gt1, col, e1)
            m2, e2 = m2n, e2n
        w1t = 1.0 / (1.0 + jnp.exp(m2 - m1))
        off = pl.ds(tile * 16, 16)
        e1_v[off] = e1
        e2_v[off] = e2
        w1_v[off] = w1t
        w2_v[off] = 1.0 - w1t
        for l in range(16):
            cnt = cnt + _b2i(lanes == _splat(e1, l))
            cnt = cnt + _b2i(lanes == _splat(e2, l))
        return cnt

    cnt = lax.fori_loop(0, TPW // 16, tile_route, jnp.zeros((16,), jnp.int32))

    cnt_v[...] = cnt
    pltpu.sync_copy(cnt_v, cntx_hbm.at[s])
    plsc.subcore_barrier()
    pltpu.sync_copy(cntx_hbm, allcnt_v)
    tot = jnp.zeros((16,), jnp.int32)
    base = jnp.zeros((16,), jnp.int32)
    for w in range(NS):
        row = allcnt_v[w]
        before = _b2i(jnp.full((16,), w, jnp.int32) < s)
        base = base + row * before
        tot = tot + row
    nblk = (tot + (BLK - 1)) >> 8
    blkcum_excl = _prefix_sum(nblk) - nblk
    goff = blkcum_excl * BLK + base

    @pl.when(s == 0)
    def _blockmap():
        for half in range(NBLK // 16):
            gv = lanes + half * 16
            acc = jnp.zeros((16,), jnp.int32)
            for e in range(N_EXPERTS):
                acc = acc + _b2i(gv >= _splat(blkcum_excl, e))
            blkexp_v[pl.ds(half * 16, 16)] = jnp.clip(acc - 1, 0, N_EXPERTS - 1)
        pltpu.sync_copy(blkexp_v, blk_e_hbm)

    def tile_pos(tile, loc):
        off = pl.ds(tile * 16, 16)
        for ev_ref, pv_ref in ((e1_v, p1_v), (e2_v, p2_v)):
            ev = ev_ref[off]
            base = _vreg_take(loc, ev)
            rank = jnp.zeros((16,), jnp.int32)
            hist = jnp.zeros((16,), jnp.int32)
            for l in range(16):
                sp = _splat(ev, l)
                rank = rank + _b2i((ev == sp) & (lanes > l))
                hist = hist + _b2i(lanes == sp)
            pv_ref[off] = base + rank
            loc = loc + hist
        return loc

    lax.fori_loop(0, TPW // 16, tile_pos, goff)

    cps = [
        pltpu.async_copy(p1_v, p1_hbm.at[pl.ds(tbase, TPW)], sem),
        pltpu.async_copy(p2_v, p2_hbm.at[pl.ds(tbase, TPW)], sem),
        pltpu.async_copy(w1_v, w1_hbm.at[pl.ds(tbase, TPW)], sem),
        pltpu.async_copy(w2_v, w2_hbm.at[pl.ds(tbase, TPW)], sem),
    ]

    half = c * (TPW // 2)
    xcp = pltpu.async_copy(x_hbm.at[pl.ds(tbase + half, TPW // 2)], rows_v, sem2)
    for i in range(TPW // 2 // 16):
        off = pl.ds(i * 16, 16)
        idx2_v[0, off] = p1_v[pl.ds(half + i * 16, 16)]
        idx2_v[1, off] = p2_v[pl.ds(half + i * 16, 16)]
    for cp in cps:
        cp.wait()
    xcp.wait()
    s1 = pltpu.async_copy(rows_v, gx_hbm.at[idx2_v.at[0]], sem)
    s2 = pltpu.async_copy(rows_v, gx_hbm.at[idx2_v.at[1]], sem)
    s1.wait()
    s2.wait()


_route_gather = functools.partial(
    pl.kernel,
    _route_gather_body,
    mesh=_mesh,
    out_type=(
        jax.ShapeDtypeStruct((NBLK,), jnp.int32),
        jax.ShapeDtypeStruct((T,), jnp.int32),
        jax.ShapeDtypeStruct((T,), jnp.int32),
        jax.ShapeDtypeStruct((T,), jnp.float32),
        jax.ShapeDtypeStruct((T,), jnp.float32),
        jax.ShapeDtypeStruct((S, D_MODEL), jnp.float32),
        jax.ShapeDtypeStruct((NS, 16), jnp.int32),
    ),
    scratch_types=[
        pltpu.VMEM((TPW * N_EXPERTS,), jnp.float32),
        pltpu.VMEM((TPW,), jnp.int32),
        pltpu.VMEM((TPW,), jnp.int32),
        pltpu.VMEM((TPW,), jnp.float32),
        pltpu.VMEM((TPW,), jnp.float32),
        pltpu.VMEM((TPW,), jnp.int32),
        pltpu.VMEM((TPW,), jnp.int32),
        pltpu.VMEM((16,), jnp.int32),
        pltpu.VMEM((NS, 16), jnp.int32),
        pltpu.VMEM((NBLK,), jnp.int32),
        pltpu.VMEM((2, TPW // 2), jnp.int32),
        pltpu.VMEM((TPW // 2, D_MODEL), jnp.float32),
        pltpu.SemaphoreType.DMA,
        pltpu.SemaphoreType.DMA,
    ],
)()


def _expert_mm_kernel(be_ref, gx_ref, we_ref, beb_ref, y_ref):
    y_ref[...] = (
        jnp.dot(gx_ref[...], we_ref[0], preferred_element_type=jnp.float32)
        + beb_ref[0]
    )


def _combine_body(
    y_hbm, p1_hbm, p2_hbm, w1_hbm, w2_hbm, out_hbm,
    p1_v, p2_v, w1_v, w2_v, a_v, b_v, o_v, sem,
):
    wid = lax.axis_index("c") * NS + lax.axis_index("s")
    tb = wid * CPW
    cps = [
        pltpu.async_copy(w1_hbm.at[pl.ds(tb, CPW)], w1_v, sem),
        pltpu.async_copy(w2_hbm.at[pl.ds(tb, CPW)], w2_v, sem),
    ]
    for ch in range(CPW // CCH):
        cps.append(
            pltpu.async_copy(p1_hbm.at[pl.ds(tb + ch * CCH, CCH)], p1_v.at[ch], sem)
        )
        cps.append(
            pltpu.async_copy(p2_hbm.at[pl.ds(tb + ch * CCH, CCH)], p2_v.at[ch], sem)
        )
    for cp in cps:
        cp.wait()
    for ch in range(CPW // CCH):
        for i in range(CCH // 16):
            off = pl.ds(i * 16, 16)
            p1_v[ch, off] = jnp.clip(p1_v[ch, off], 0, S - 1)
            p2_v[ch, off] = jnp.clip(p2_v[ch, off], 0, S - 1)

    def chunk_body(ch, _):
        ga = pltpu.async_copy(y_hbm.at[p1_v.at[ch]], a_v, sem)
        gb = pltpu.async_copy(y_hbm.at[p2_v.at[ch]], b_v, sem)
        ga.wait()
        gb.wait()

        def grp_body(g, _):
            w16_1 = w1_v[pl.ds(ch * CCH + g * 16, 16)]
            w16_2 = w2_v[pl.ds(ch * CCH + g * 16, 16)]
            for l in range(16):
                i = g * 16 + l
                lidx = jnp.full((16,), l, jnp.int32)
                w1b = _vreg_take(w16_1, lidx)
                w2b = _vreg_take(w16_2, lidx)
                for v in range(D_MODEL // 16):
                    off = pl.ds(v * 16, 16)
                    o_v[i, off] = w1b * a_v[i, off] + w2b * b_v[i, off]
            return 0

        lax.fori_loop(0, CCH // 16, grp_body, 0)
        pltpu.sync_copy(o_v, out_hbm.at[pl.ds(tb + ch * CCH, CCH)])
        return 0

    lax.fori_loop(0, CPW // CCH, chunk_body, 0)


_combine = functools.partial(
    pl.kernel,
    _combine_body,
    mesh=_mesh,
    out_type=jax.ShapeDtypeStruct((T, D_MODEL), jnp.float32),
    scratch_types=[
        pltpu.VMEM((CPW // CCH, CCH), jnp.int32),
        pltpu.VMEM((CPW // CCH, CCH), jnp.int32),
        pltpu.VMEM((CPW,), jnp.float32),
        pltpu.VMEM((CPW,), jnp.float32),
        pltpu.VMEM((CCH, D_MODEL), jnp.float32),
        pltpu.VMEM((CCH, D_MODEL), jnp.float32),
        pltpu.VMEM((CCH, D_MODEL), jnp.float32),
        pltpu.SemaphoreType.DMA,
    ],
)()


def kernel(inputs, Wg, bg, We, be):
    x2 = inputs.reshape(T, D_MODEL)
    bg2 = bg.reshape(1, N_EXPERTS)
    be3 = be.reshape(N_EXPERTS, 1, D_MODEL)

    logits = pl.pallas_call(
        _gate_kernel,
        out_shape=jax.ShapeDtypeStruct((N_EXPERTS, T), jnp.float32),
    )(x2, Wg, bg2)

    blk_e, p1, p2, w1, w2, gx, _cx = _route_gather(logits.reshape(-1), x2)

    grid_spec = pltpu.PrefetchScalarGridSpec(
        num_scalar_prefetch=1,
        grid=(NBLK,),
        in_specs=[
            pl.BlockSpec((BLK, D_MODEL), lambda g, be_r: (g, 0)),
            pl.BlockSpec((1, D_MODEL, D_MODEL), lambda g, be_r: (be_r[g], 0, 0)),
            pl.BlockSpec((1, 1, D_MODEL), lambda g, be_r: (be_r[g], 0, 0)),
        ],
        out_specs=pl.BlockSpec((BLK, D_MODEL), lambda g, be_r: (g, 0)),
    )
    y = pl.pallas_call(
        _expert_mm_kernel,
        grid_spec=grid_spec,
        out_shape=jax.ShapeDtypeStruct((S, D_MODEL), jnp.float32),
    )(blk_e, gx, We, be3)

    out = _combine(y, p1, p2, w1, w2)
    return out.reshape(inputs.shape)
